# baseline (speedup 1.0000x reference)
import functools

import numpy as np

import jax
import jax.numpy as jnp
from jax import lax
from jax.experimental import pallas as pl
from jax.experimental.pallas import tpu as pltpu
from jax.experimental.pallas import tpu_sc as plsc

VOCAB = 100000
D = 64
B = 1024
S = 200

NC = 2
NS = 16
NW = NC * NS
ROWS_PER_W = B // NW
CHUNK_ROWS = 4
N_CHUNKS = ROWS_PER_W // CHUNK_ROWS
IDX = CHUNK_ROWS * S
L = 16
DG = D // L


def _pool_body(xp_hbm, table_hbm, h_hbm, idx_0, idx_1,
               rows_0, rows_1, out_v, sem_g0, sem_g1):
    i32 = jnp.int32
    wid = lax.axis_index("s") * i32(NC) + lax.axis_index("c")
    base_row = wid * i32(ROWS_PER_W)
    sems = (sem_g0, sem_g1)
    idxs = (idx_0, idx_1)
    rows = (rows_0, rows_1)

    def fetch_idx(c, buf):
        start = (base_row + i32(c * CHUNK_ROWS)) * i32(S)
        pltpu.sync_copy(xp_hbm.at[pl.ds(start, IDX)], idxs[buf])

    def start_gather(buf):
        return pltpu.async_copy(table_hbm.at[idxs[buf]],
                                rows[buf], sems[buf])

    fetch_idx(0, 0)
    handles = {0: start_gather(0)}
    for c in range(N_CHUNKS):
        buf = c % 2
        if c + 1 < N_CHUNKS:
            nb = (c + 1) % 2
            fetch_idx(c + 1, nb)
            handles[nb] = start_gather(nb)
        handles[buf].wait()
        for r in range(CHUNK_ROWS):
            def t_body(t, accs):
                base = i32(r * S) + t
                return tuple(accs[j] + rows[buf][base, pl.ds(j * L, L)]
                             for j in range(DG))
            zeros = tuple(jnp.zeros((L,), jnp.float32) for _ in range(DG))
            accs = lax.fori_loop(i32(0), i32(S), t_body, zeros)
            for j in range(DG):
                out_v[c * CHUNK_ROWS + r, pl.ds(j * L, L)] = (
                    accs[j] * (1.0 / S))
    pltpu.sync_copy(out_v, h_hbm.at[pl.ds(base_row, ROWS_PER_W)])


@jax.jit
def _pool(x_flat, table):
    mesh = plsc.VectorSubcoreMesh(core_axis_name="c", subcore_axis_name="s",
                                  num_cores=NC, num_subcores=NS)
    return pl.kernel(
        _pool_body,
        out_type=jax.ShapeDtypeStruct((B, D), jnp.float32),
        mesh=mesh,
        scratch_types=[
            pltpu.VMEM((IDX,), jnp.int32),
            pltpu.VMEM((IDX,), jnp.int32),
            pltpu.VMEM((IDX, D), jnp.float32),
            pltpu.VMEM((IDX, D), jnp.float32),
            pltpu.VMEM((ROWS_PER_W, D), jnp.float32),
            pltpu.SemaphoreType.DMA,
            pltpu.SemaphoreType.DMA,
        ],
        compiler_params=pltpu.CompilerParams(use_tc_tiling_on_sc=False),
    )(x_flat, table)


BN = 4096
_z = np.int32(0)


def _proj_body(h_ref, w_ref, b_ref, o_ref):
    o_ref[...] = lax.dot_general(
        h_ref[...], w_ref[...], (((1,), (1,)), ((), ())),
        preferred_element_type=jnp.float32) + b_ref[...]


@jax.jit
def _proj(h, W, b2d):
    grid = pl.cdiv(VOCAB, BN)
    return pl.pallas_call(
        _proj_body,
        grid=(grid,),
        in_specs=[
            pl.BlockSpec((B, D), lambda i: (_z, _z)),
            pl.BlockSpec((BN, D), lambda i: (i, _z)),
            pl.BlockSpec((1, BN), lambda i: (_z, i)),
        ],
        out_specs=pl.BlockSpec((B, BN), lambda i: (_z, i)),
        out_shape=jax.ShapeDtypeStruct((B, VOCAB), jnp.float32),
    )(h, W, b2d)


def kernel(x, table, W, b):
    x_flat = x.reshape(-1).astype(jnp.int32)
    h = _pool(x_flat, table)
    return _proj(h, W, b.reshape(1, VOCAB))
